# baseline (speedup 1.0000x reference)
import functools

import jax
import jax.numpy as jnp
from jax import lax
from jax.experimental import pallas as pl
from jax.experimental.pallas import tpu as pltpu
from jax.experimental.pallas import tpu_sc as plsc

N_NODES = 10000
D_FEAT = 128
HIDDEN = 256
N_CLASSES = 40
N_EDGES = 320000

NC = 2
NS = 16
NW = NC * NS
CH = 128
NCH = 80
BLK = 8
NBLK = NCH // BLK
EPT = NCH * CH
E_PAD = NW * EPT
NPAD = 10112
ROWS_PT = NPAD // NS
DEGW = 128
L2W = 128

ROW_BLOCK = 1264
FIN_BLOCK = 400


def _sc_mesh():
    return plsc.VectorSubcoreMesh(core_axis_name="c", subcore_axis_name="s")


@functools.partial(
    pl.kernel,
    mesh=_sc_mesh(),
    out_type=jax.ShapeDtypeStruct((NC, NS, ROWS_PT, DEGW), jnp.float32),
    scratch_types=[
        pltpu.VMEM((NCH, CH), jnp.int32),
        pltpu.VMEM((CH, DEGW), jnp.float32),
        pltpu.VMEM_SHARED((NPAD, DEGW), jnp.float32),
        pltpu.SemaphoreType.DMA,
    ],
)
def _deg_kernel(dst_hbm, ones_hbm, zero_hbm, out_hbm, idx_v, ones_v, deg_sh, sem):
    c = lax.axis_index("c")
    s = lax.axis_index("s")
    wid = s * NC + c
    pltpu.sync_copy(zero_hbm, deg_sh.at[pl.ds(s * ROWS_PT, ROWS_PT)])
    pltpu.sync_copy(ones_hbm, ones_v)
    pltpu.sync_copy(dst_hbm.at[wid], idx_v)
    plsc.subcore_barrier()

    def body(j, carry):
        pltpu.async_copy(ones_v, deg_sh.at[idx_v.at[j]], sem, add=True)
        return carry

    lax.fori_loop(0, NCH, body, 0)

    def drain(j, carry):
        pltpu.make_async_copy(ones_v, deg_sh.at[idx_v.at[0]], sem).wait()
        return carry

    lax.fori_loop(0, NCH, drain, 0)
    plsc.subcore_barrier()
    pltpu.sync_copy(deg_sh.at[pl.ds(s * ROWS_PT, ROWS_PT)], out_hbm.at[c, s])


def _make_agg_kernel(D):
    @functools.partial(
        pl.kernel,
        mesh=_sc_mesh(),
        out_type=jax.ShapeDtypeStruct((NC, NS, ROWS_PT, D), jnp.float32),
        scratch_types=[
            pltpu.VMEM((BLK, CH), jnp.int32),
            pltpu.VMEM((BLK, CH), jnp.int32),
            pltpu.VMEM((NCH, CH), jnp.int32),
            pltpu.VMEM((CH, D), jnp.float32),
            pltpu.VMEM((CH, D), jnp.float32),
            pltpu.VMEM_SHARED((NPAD, D), jnp.float32),
            pltpu.SemaphoreType.DMA,
            pltpu.SemaphoreType.DMA,
            pltpu.SemaphoreType.DMA,
            pltpu.SemaphoreType.DMA,
            pltpu.SemaphoreType.DMA,
            pltpu.SemaphoreType.DMA,
        ],
    )
    def _agg(g_hbm, src_hbm, dst_hbm, zero_hbm, out_hbm,
             si0_v, si1_v, di_v, rows0_v, rows1_v, acc_sh,
             sem_s0, sem_s1, sem_g0, sem_g1, sem_c0, sem_c1):
        c = lax.axis_index("c")
        s = lax.axis_index("s")
        wid = s * NC + c
        si = (si0_v, si1_v)
        sem_s = (sem_s0, sem_s1)
        rows = (rows0_v, rows1_v)
        sem_g = (sem_g0, sem_g1)
        sem_c = (sem_c0, sem_c1)

        pltpu.async_copy(src_hbm.at[wid, pl.ds(0, BLK)], si0_v, sem_s0)
        pltpu.async_copy(src_hbm.at[wid, pl.ds(BLK, BLK)], si1_v, sem_s1)
        pltpu.sync_copy(dst_hbm.at[wid], di_v)
        pltpu.sync_copy(zero_hbm, acc_sh.at[pl.ds(s * ROWS_PT, ROWS_PT)])
        plsc.subcore_barrier()

        pltpu.make_async_copy(
            src_hbm.at[wid, pl.ds(0, BLK)], si0_v, sem_s0).wait()
        pltpu.async_copy(g_hbm.at[si0_v.at[0]], rows0_v, sem_g0)

        def half_body(p, half):
            b = 2 * p + half
            sbuf = si[half]
            for k in range(BLK):
                j = b * BLK + k
                rp = k % 2
                gp = (k + 1) % 2

                if k == 0 and half == 0:
                    @pl.when(b > 0)
                    def _drain_prev():
                        pltpu.make_async_copy(
                            rows[gp], acc_sh.at[di_v.at[0]], sem_c[gp]).wait()
                else:
                    pltpu.make_async_copy(
                        rows[gp], acc_sh.at[di_v.at[0]], sem_c[gp]).wait()

                if k < BLK - 1:
                    pltpu.async_copy(g_hbm.at[sbuf.at[k + 1]],
                                     rows[gp], sem_g[gp])
                else:
                    nhalf = 1 - half

                    @pl.when(b + 1 < NBLK)
                    def _next_block():
                        pltpu.make_async_copy(
                            src_hbm.at[wid, pl.ds(0, BLK)],
                            si[nhalf], sem_s[nhalf]).wait()
                        pltpu.async_copy(g_hbm.at[si[nhalf].at[0]],
                                         rows[gp], sem_g[gp])

                pltpu.make_async_copy(g_hbm.at[sbuf.at[k]],
                                      rows[rp], sem_g[rp]).wait()
                pltpu.async_copy(rows[rp], acc_sh.at[di_v.at[j]],
                                 sem_c[rp], add=True)

            @pl.when(b + 2 < NBLK)
            def _prefetch():
                pltpu.async_copy(
                    src_hbm.at[wid, pl.ds((b + 2) * BLK, BLK)],
                    sbuf, sem_s[half])

        def pair_body(p, carry):
            half_body(p, 0)
            half_body(p, 1)
            return carry

        lax.fori_loop(0, NBLK // 2, pair_body, 0)
        pltpu.make_async_copy(rows[1], acc_sh.at[di_v.at[0]], sem_c[1]).wait()
        plsc.subcore_barrier()
        pltpu.sync_copy(acc_sh.at[pl.ds(s * ROWS_PT, ROWS_PT)], out_hbm.at[c, s])

    return _agg


_agg_kernel_l1 = _make_agg_kernel(D_FEAT)
_agg_kernel_l2 = _agg_kernel_l1


def _prep_body(deg_ref, x_ref, g_ref, dinv_ref):
    h = deg_ref[...]
    deg = h[0, :, 0:1] + h[1, :, 0:1] + 1.0
    dinv = lax.rsqrt(deg)
    dinv_ref[...] = dinv
    g_ref[...] = x_ref[...] * dinv


def _prep_call(deg_parts, xp):
    grid = NPAD // ROW_BLOCK
    return pl.pallas_call(
        _prep_body,
        grid=(grid,),
        in_specs=[
            pl.BlockSpec((NC, ROW_BLOCK, DEGW), lambda i: (0, i, 0)),
            pl.BlockSpec((ROW_BLOCK, D_FEAT), lambda i: (i, 0)),
        ],
        out_specs=[
            pl.BlockSpec((ROW_BLOCK, D_FEAT), lambda i: (i, 0)),
            pl.BlockSpec((ROW_BLOCK, 1), lambda i: (i, 0)),
        ],
        out_shape=[
            jax.ShapeDtypeStruct((NPAD, D_FEAT), jnp.float32),
            jax.ShapeDtypeStruct((NPAD, 1), jnp.float32),
        ],
    )(deg_parts, xp)


def _mid_body(acc_ref, g_ref, dinv_ref, w1_ref, b1_ref, w2_ref, g2_ref):
    a = acc_ref[...]
    dinv = dinv_ref[...]
    agg = (a[0] + a[1] + g_ref[...]) * dinv
    h1 = jnp.dot(agg, w1_ref[...], preferred_element_type=jnp.float32,
                 precision=lax.Precision.HIGHEST)
    h1 = jnp.maximum(h1 + b1_ref[...], 0.0)
    p = jnp.dot(h1, w2_ref[...], preferred_element_type=jnp.float32,
                precision=lax.Precision.HIGHEST)
    g2_ref[...] = p * dinv


def _mid_call(acc_parts, g, dinv, W1, b1, W2p):
    grid = NPAD // ROW_BLOCK
    return pl.pallas_call(
        _mid_body,
        grid=(grid,),
        in_specs=[
            pl.BlockSpec((NC, ROW_BLOCK, D_FEAT), lambda i: (0, i, 0)),
            pl.BlockSpec((ROW_BLOCK, D_FEAT), lambda i: (i, 0)),
            pl.BlockSpec((ROW_BLOCK, 1), lambda i: (i, 0)),
            pl.BlockSpec((D_FEAT, HIDDEN), lambda i: (0, 0)),
            pl.BlockSpec((1, HIDDEN), lambda i: (0, 0)),
            pl.BlockSpec((HIDDEN, L2W), lambda i: (0, 0)),
        ],
        out_specs=pl.BlockSpec((ROW_BLOCK, L2W), lambda i: (i, 0)),
        out_shape=jax.ShapeDtypeStruct((NPAD, L2W), jnp.float32),
    )(acc_parts, g, dinv, W1, b1, W2p)


def _fin_body(acc_ref, g2_ref, dinv_ref, b2_ref, out_ref):
    a = acc_ref[...]
    z = (a[0] + a[1] + g2_ref[...]) * dinv_ref[...]
    z = z[:, :N_CLASSES] + b2_ref[...]
    m = jnp.max(z, axis=1, keepdims=True)
    zm = z - m
    lse = jnp.log(jnp.sum(jnp.exp(zm), axis=1, keepdims=True))
    out_ref[...] = zm - lse


def _fin_call(acc2_parts, g2, dinv, b2):
    grid = N_NODES // FIN_BLOCK
    return pl.pallas_call(
        _fin_body,
        grid=(grid,),
        in_specs=[
            pl.BlockSpec((NC, FIN_BLOCK, L2W), lambda i: (0, i, 0)),
            pl.BlockSpec((FIN_BLOCK, L2W), lambda i: (i, 0)),
            pl.BlockSpec((FIN_BLOCK, 1), lambda i: (i, 0)),
            pl.BlockSpec((1, N_CLASSES), lambda i: (0, 0)),
        ],
        out_specs=pl.BlockSpec((FIN_BLOCK, N_CLASSES), lambda i: (i, 0)),
        out_shape=jax.ShapeDtypeStruct((N_NODES, N_CLASSES), jnp.float32),
    )(acc2_parts, g2, dinv, b2)


def kernel(x, edge_index, W1, b1, W2, b2):
    n_extra = E_PAD - N_EDGES
    pad_ids = jnp.arange(n_extra, dtype=jnp.int32)
    src = jnp.concatenate(
        [edge_index[0].astype(jnp.int32), pad_ids % N_NODES]).reshape(NW, NCH, CH)
    dst = jnp.concatenate(
        [edge_index[1].astype(jnp.int32),
         N_NODES + pad_ids % (NPAD - N_NODES)]).reshape(NW, NCH, CH)

    xp = jnp.pad(x, ((0, NPAD - N_NODES), (0, 0)))
    ones_rows = jnp.ones((CH, DEGW), jnp.float32)
    zero_deg = jnp.zeros((ROWS_PT, DEGW), jnp.float32)
    zero_128 = jnp.zeros((ROWS_PT, D_FEAT), jnp.float32)
    W2p = jnp.pad(W2, ((0, 0), (0, L2W - N_CLASSES)))
    b1r = b1.reshape(1, HIDDEN)
    b2r = b2.reshape(1, N_CLASSES)

    deg_parts = _deg_kernel(dst, ones_rows, zero_deg)
    deg_parts = deg_parts.reshape(NC, NPAD, DEGW)

    g, dinv = _prep_call(deg_parts, xp)

    acc = _agg_kernel_l1(g, src, dst, zero_128)
    acc = acc.reshape(NC, NPAD, D_FEAT)

    g2 = _mid_call(acc, g, dinv, W1, b1r, W2p)

    acc2 = _agg_kernel_l2(g2, src, dst, zero_128)
    acc2 = acc2.reshape(NC, NPAD, L2W)

    return _fin_call(acc2, g2, dinv, b2r)


# speedup vs baseline: 1.1725x; 1.1725x over previous
import functools

import jax
import jax.numpy as jnp
from jax import lax
from jax.experimental import pallas as pl
from jax.experimental.pallas import tpu as pltpu
from jax.experimental.pallas import tpu_sc as plsc

N_NODES = 10000
D_FEAT = 128
HIDDEN = 256
N_CLASSES = 40
N_EDGES = 320000

NC = 2
NS = 16
NW = NC * NS
CH = 128
NCH = 80
BLK = 8
NBLK = NCH // BLK
EPT = NCH * CH
E_PAD = NW * EPT
NPAD = 10112
ROWS_PT = NPAD // NS
DEGW = 128
L2W = 128

ROW_BLOCK = 1264
FIN_BLOCK = 400


def _sc_mesh():
    return plsc.VectorSubcoreMesh(core_axis_name="c", subcore_axis_name="s")


@functools.partial(
    pl.kernel,
    mesh=_sc_mesh(),
    compiler_params=pltpu.CompilerParams(needs_layout_passes=False),
    out_type=jax.ShapeDtypeStruct((NC, NS, NPAD), jnp.float32),
    scratch_types=[
        pltpu.VMEM((NCH, CH), jnp.int32),
        pltpu.VMEM((NPAD,), jnp.float32),
    ],
)
def _deg_kernel(dst_hbm, zero_hbm, out_hbm, idx_v, hist_v):
    c = lax.axis_index("c")
    s = lax.axis_index("s")
    wid = s * NC + c
    pltpu.sync_copy(zero_hbm, hist_v)
    pltpu.sync_copy(dst_hbm.at[wid], idx_v)
    ones = jnp.full((16,), 1.0, jnp.float32)

    def body(j, carry):
        for t in range(CH // 16):
            idx = idx_v[j, pl.ds(t * 16, 16)]
            plsc.addupdate_scatter(hist_v, [idx], ones)
        return carry

    lax.fori_loop(0, NCH, body, 0)
    pltpu.sync_copy(hist_v, out_hbm.at[c, s])


def _make_agg_kernel(D):
    @functools.partial(
        pl.kernel,
        mesh=_sc_mesh(),
        out_type=jax.ShapeDtypeStruct((NC, NS, ROWS_PT, D), jnp.float32),
        scratch_types=[
            pltpu.VMEM((BLK, CH), jnp.int32),
            pltpu.VMEM((BLK, CH), jnp.int32),
            pltpu.VMEM((NCH, CH), jnp.int32),
            pltpu.VMEM((CH, D), jnp.float32),
            pltpu.VMEM((CH, D), jnp.float32),
            pltpu.VMEM_SHARED((NPAD, D), jnp.float32),
            pltpu.SemaphoreType.DMA,
            pltpu.SemaphoreType.DMA,
            pltpu.SemaphoreType.DMA,
            pltpu.SemaphoreType.DMA,
            pltpu.SemaphoreType.DMA,
            pltpu.SemaphoreType.DMA,
        ],
    )
    def _agg(g_hbm, src_hbm, dst_hbm, zero_hbm, out_hbm,
             si0_v, si1_v, di_v, rows0_v, rows1_v, acc_sh,
             sem_s0, sem_s1, sem_g0, sem_g1, sem_c0, sem_c1):
        c = lax.axis_index("c")
        s = lax.axis_index("s")
        wid = s * NC + c
        si = (si0_v, si1_v)
        sem_s = (sem_s0, sem_s1)
        rows = (rows0_v, rows1_v)
        sem_g = (sem_g0, sem_g1)
        sem_c = (sem_c0, sem_c1)

        pltpu.async_copy(src_hbm.at[wid, pl.ds(0, BLK)], si0_v, sem_s0)
        pltpu.async_copy(src_hbm.at[wid, pl.ds(BLK, BLK)], si1_v, sem_s1)
        pltpu.sync_copy(dst_hbm.at[wid], di_v)
        pltpu.sync_copy(zero_hbm, acc_sh.at[pl.ds(s * ROWS_PT, ROWS_PT)])
        plsc.subcore_barrier()

        pltpu.make_async_copy(
            src_hbm.at[wid, pl.ds(0, BLK)], si0_v, sem_s0).wait()
        pltpu.async_copy(g_hbm.at[si0_v.at[0]], rows0_v, sem_g0)

        def half_body(p, half):
            b = 2 * p + half
            sbuf = si[half]
            for k in range(BLK):
                j = b * BLK + k
                rp = k % 2
                gp = (k + 1) % 2

                if k == 0 and half == 0:
                    @pl.when(b > 0)
                    def _drain_prev():
                        pltpu.make_async_copy(
                            rows[gp], acc_sh.at[di_v.at[0]], sem_c[gp]).wait()
                else:
                    pltpu.make_async_copy(
                        rows[gp], acc_sh.at[di_v.at[0]], sem_c[gp]).wait()

                if k < BLK - 1:
                    pltpu.async_copy(g_hbm.at[sbuf.at[k + 1]],
                                     rows[gp], sem_g[gp])
                else:
                    nhalf = 1 - half

                    @pl.when(b + 1 < NBLK)
                    def _next_block():
                        pltpu.make_async_copy(
                            src_hbm.at[wid, pl.ds(0, BLK)],
                            si[nhalf], sem_s[nhalf]).wait()
                        pltpu.async_copy(g_hbm.at[si[nhalf].at[0]],
                                         rows[gp], sem_g[gp])

                pltpu.make_async_copy(g_hbm.at[sbuf.at[k]],
                                      rows[rp], sem_g[rp]).wait()
                pltpu.async_copy(rows[rp], acc_sh.at[di_v.at[j]],
                                 sem_c[rp], add=True)

            @pl.when(b + 2 < NBLK)
            def _prefetch():
                pltpu.async_copy(
                    src_hbm.at[wid, pl.ds((b + 2) * BLK, BLK)],
                    sbuf, sem_s[half])

        def pair_body(p, carry):
            half_body(p, 0)
            half_body(p, 1)
            return carry

        lax.fori_loop(0, NBLK // 2, pair_body, 0)
        pltpu.make_async_copy(rows[1], acc_sh.at[di_v.at[0]], sem_c[1]).wait()
        plsc.subcore_barrier()
        pltpu.sync_copy(acc_sh.at[pl.ds(s * ROWS_PT, ROWS_PT)], out_hbm.at[c, s])

    return _agg


_agg_kernel_l1 = _make_agg_kernel(D_FEAT)
_agg_kernel_l2 = _agg_kernel_l1


def _prep_body(deg_ref, x_ref, g_ref, dinv_ref):
    h = deg_ref[...]
    deg = jnp.sum(h, axis=1, keepdims=True) + 1.0
    dinv = lax.rsqrt(deg)
    dinv_ref[...] = dinv
    g_ref[...] = x_ref[...] * dinv


def _prep_call(deg_parts, xp):
    grid = NPAD // ROW_BLOCK
    return pl.pallas_call(
        _prep_body,
        grid=(grid,),
        in_specs=[
            pl.BlockSpec((ROW_BLOCK, NW), lambda i: (i, 0)),
            pl.BlockSpec((ROW_BLOCK, D_FEAT), lambda i: (i, 0)),
        ],
        out_specs=[
            pl.BlockSpec((ROW_BLOCK, D_FEAT), lambda i: (i, 0)),
            pl.BlockSpec((ROW_BLOCK, 1), lambda i: (i, 0)),
        ],
        out_shape=[
            jax.ShapeDtypeStruct((NPAD, D_FEAT), jnp.float32),
            jax.ShapeDtypeStruct((NPAD, 1), jnp.float32),
        ],
    )(deg_parts, xp)


def _mid_body(acc_ref, g_ref, dinv_ref, w1_ref, b1_ref, w2_ref, g2_ref):
    a = acc_ref[...]
    dinv = dinv_ref[...]
    agg = (a[0] + a[1] + g_ref[...]) * dinv
    h1 = jnp.dot(agg, w1_ref[...], preferred_element_type=jnp.float32,
                 precision=lax.Precision.HIGHEST)
    h1 = jnp.maximum(h1 + b1_ref[...], 0.0)
    p = jnp.dot(h1, w2_ref[...], preferred_element_type=jnp.float32,
                precision=lax.Precision.HIGHEST)
    g2_ref[...] = p * dinv


def _mid_call(acc_parts, g, dinv, W1, b1, W2p):
    grid = NPAD // ROW_BLOCK
    return pl.pallas_call(
        _mid_body,
        grid=(grid,),
        in_specs=[
            pl.BlockSpec((NC, ROW_BLOCK, D_FEAT), lambda i: (0, i, 0)),
            pl.BlockSpec((ROW_BLOCK, D_FEAT), lambda i: (i, 0)),
            pl.BlockSpec((ROW_BLOCK, 1), lambda i: (i, 0)),
            pl.BlockSpec((D_FEAT, HIDDEN), lambda i: (0, 0)),
            pl.BlockSpec((1, HIDDEN), lambda i: (0, 0)),
            pl.BlockSpec((HIDDEN, L2W), lambda i: (0, 0)),
        ],
        out_specs=pl.BlockSpec((ROW_BLOCK, L2W), lambda i: (i, 0)),
        out_shape=jax.ShapeDtypeStruct((NPAD, L2W), jnp.float32),
    )(acc_parts, g, dinv, W1, b1, W2p)


def _fin_body(acc_ref, g2_ref, dinv_ref, b2_ref, out_ref):
    a = acc_ref[...]
    z = (a[0] + a[1] + g2_ref[...]) * dinv_ref[...]
    z = z[:, :N_CLASSES] + b2_ref[...]
    m = jnp.max(z, axis=1, keepdims=True)
    zm = z - m
    lse = jnp.log(jnp.sum(jnp.exp(zm), axis=1, keepdims=True))
    out_ref[...] = zm - lse


def _fin_call(acc2_parts, g2, dinv, b2):
    grid = N_NODES // FIN_BLOCK
    return pl.pallas_call(
        _fin_body,
        grid=(grid,),
        in_specs=[
            pl.BlockSpec((NC, FIN_BLOCK, L2W), lambda i: (0, i, 0)),
            pl.BlockSpec((FIN_BLOCK, L2W), lambda i: (i, 0)),
            pl.BlockSpec((FIN_BLOCK, 1), lambda i: (i, 0)),
            pl.BlockSpec((1, N_CLASSES), lambda i: (0, 0)),
        ],
        out_specs=pl.BlockSpec((FIN_BLOCK, N_CLASSES), lambda i: (i, 0)),
        out_shape=jax.ShapeDtypeStruct((N_NODES, N_CLASSES), jnp.float32),
    )(acc2_parts, g2, dinv, b2)


def kernel(x, edge_index, W1, b1, W2, b2):
    n_extra = E_PAD - N_EDGES
    pad_ids = jnp.arange(n_extra, dtype=jnp.int32)
    src = jnp.concatenate(
        [edge_index[0].astype(jnp.int32), pad_ids % N_NODES]).reshape(NW, NCH, CH)
    dst = jnp.concatenate(
        [edge_index[1].astype(jnp.int32),
         N_NODES + pad_ids % (NPAD - N_NODES)]).reshape(NW, NCH, CH)

    xp = jnp.pad(x, ((0, NPAD - N_NODES), (0, 0)))
    zero_hist = jnp.zeros((NPAD,), jnp.float32)
    zero_128 = jnp.zeros((ROWS_PT, D_FEAT), jnp.float32)
    W2p = jnp.pad(W2, ((0, 0), (0, L2W - N_CLASSES)))
    b1r = b1.reshape(1, HIDDEN)
    b2r = b2.reshape(1, N_CLASSES)

    deg_parts = _deg_kernel(dst, zero_hist)
    deg_t = deg_parts.reshape(NW, NPAD).T

    g, dinv = _prep_call(deg_t, xp)

    acc = _agg_kernel_l1(g, src, dst, zero_128)
    acc = acc.reshape(NC, NPAD, D_FEAT)

    g2 = _mid_call(acc, g, dinv, W1, b1r, W2p)

    acc2 = _agg_kernel_l2(g2, src, dst, zero_128)
    acc2 = acc2.reshape(NC, NPAD, L2W)

    return _fin_call(acc2, g2, dinv, b2r)


# speedup vs baseline: 1.1737x; 1.0011x over previous
import functools

import jax
import jax.numpy as jnp
from jax import lax
from jax.experimental import pallas as pl
from jax.experimental.pallas import tpu as pltpu
from jax.experimental.pallas import tpu_sc as plsc

N_NODES = 10000
D_FEAT = 128
HIDDEN = 256
N_CLASSES = 40
N_EDGES = 320000

NC = 2
NS = 16
NW = NC * NS
CH = 128
NCH = 80
BLK = 8
NBLK = NCH // BLK
EPT = NCH * CH
E_PAD = NW * EPT
NPAD = 10112
ROWS_PT = NPAD // NS
L2W = 128

ROW_BLOCK = 1264
FIN_BLOCK = 400


def _sc_mesh():
    return plsc.VectorSubcoreMesh(core_axis_name="c", subcore_axis_name="s")


@functools.partial(
    pl.kernel,
    mesh=_sc_mesh(),
    compiler_params=pltpu.CompilerParams(needs_layout_passes=False),
    out_type=jax.ShapeDtypeStruct((NC, NS, NPAD), jnp.float32),
    scratch_types=[
        pltpu.VMEM((NCH, CH), jnp.int32),
        pltpu.VMEM((NPAD,), jnp.float32),
    ],
)
def _deg_kernel(dst_hbm, zero_hbm, out_hbm, idx_v, hist_v):
    c = lax.axis_index("c")
    s = lax.axis_index("s")
    wid = s * NC + c
    pltpu.sync_copy(zero_hbm, hist_v)
    pltpu.sync_copy(dst_hbm.at[wid], idx_v)
    ones = jnp.full((16,), 1.0, jnp.float32)

    def body(j, carry):
        for t in range(CH // 16):
            idx = idx_v[j, pl.ds(t * 16, 16)]
            plsc.addupdate_scatter(hist_v, [idx], ones)
        return carry

    lax.fori_loop(0, NCH, body, 0)
    pltpu.sync_copy(hist_v, out_hbm.at[c, s])


def _make_agg_kernel(D):
    @functools.partial(
        pl.kernel,
        mesh=_sc_mesh(),
        out_type=jax.ShapeDtypeStruct((NC, NS, ROWS_PT, D), jnp.float32),
        scratch_types=[
            pltpu.VMEM((BLK, CH), jnp.int32),
            pltpu.VMEM((BLK, CH), jnp.int32),
            pltpu.VMEM((NCH, CH), jnp.int32),
            pltpu.VMEM((CH, D), jnp.float32),
            pltpu.VMEM((CH, D), jnp.float32),
            pltpu.VMEM_SHARED((NPAD, D), jnp.float32),
            pltpu.SemaphoreType.DMA,
            pltpu.SemaphoreType.DMA,
            pltpu.SemaphoreType.DMA,
            pltpu.SemaphoreType.DMA,
            pltpu.SemaphoreType.DMA,
            pltpu.SemaphoreType.DMA,
        ],
    )
    def _agg(g_hbm, src_hbm, dst_hbm, zero_hbm, out_hbm,
             si0_v, si1_v, di_v, rows0_v, rows1_v, acc_sh,
             sem_s0, sem_s1, sem_g0, sem_g1, sem_c0, sem_c1):
        c = lax.axis_index("c")
        s = lax.axis_index("s")
        wid = s * NC + c
        si = (si0_v, si1_v)
        sem_s = (sem_s0, sem_s1)
        rows = (rows0_v, rows1_v)
        sem_g = (sem_g0, sem_g1)
        sem_c = (sem_c0, sem_c1)

        pltpu.async_copy(src_hbm.at[wid, pl.ds(0, BLK)], si0_v, sem_s0)
        pltpu.async_copy(src_hbm.at[wid, pl.ds(BLK, BLK)], si1_v, sem_s1)
        pltpu.sync_copy(dst_hbm.at[wid], di_v)
        pltpu.sync_copy(zero_hbm, acc_sh.at[pl.ds(s * ROWS_PT, ROWS_PT)])
        plsc.subcore_barrier()

        pltpu.make_async_copy(
            src_hbm.at[wid, pl.ds(0, BLK)], si0_v, sem_s0).wait()
        pltpu.async_copy(g_hbm.at[si0_v.at[0]], rows0_v, sem_g0)

        def half_body(p, half):
            b = 2 * p + half
            sbuf = si[half]
            for k in range(BLK):
                j = b * BLK + k
                rp = k % 2
                gp = (k + 1) % 2

                if k == 0 and half == 0:
                    @pl.when(b > 0)
                    def _drain_prev():
                        pltpu.make_async_copy(
                            rows[gp], acc_sh.at[di_v.at[0]], sem_c[gp]).wait()
                else:
                    pltpu.make_async_copy(
                        rows[gp], acc_sh.at[di_v.at[0]], sem_c[gp]).wait()

                if k < BLK - 1:
                    pltpu.async_copy(g_hbm.at[sbuf.at[k + 1]],
                                     rows[gp], sem_g[gp])
                else:
                    nhalf = 1 - half

                    @pl.when(b + 1 < NBLK)
                    def _next_block():
                        pltpu.make_async_copy(
                            src_hbm.at[wid, pl.ds(0, BLK)],
                            si[nhalf], sem_s[nhalf]).wait()
                        pltpu.async_copy(g_hbm.at[si[nhalf].at[0]],
                                         rows[gp], sem_g[gp])

                pltpu.make_async_copy(g_hbm.at[sbuf.at[k]],
                                      rows[rp], sem_g[rp]).wait()
                pltpu.async_copy(rows[rp], acc_sh.at[di_v.at[j]],
                                 sem_c[rp], add=True)

            @pl.when(b + 2 < NBLK)
            def _prefetch():
                pltpu.async_copy(
                    src_hbm.at[wid, pl.ds((b + 2) * BLK, BLK)],
                    sbuf, sem_s[half])

        def pair_body(p, carry):
            half_body(p, 0)
            half_body(p, 1)
            return carry

        lax.fori_loop(0, NBLK // 2, pair_body, 0)
        pltpu.make_async_copy(rows[1], acc_sh.at[di_v.at[0]], sem_c[1]).wait()
        plsc.subcore_barrier()
        pltpu.sync_copy(acc_sh.at[pl.ds(s * ROWS_PT, ROWS_PT)], out_hbm.at[c, s])

    return _agg


_agg_kernel_l1 = _make_agg_kernel(D_FEAT)
_agg_kernel_l2 = _agg_kernel_l1


def _prep_body(deg_ref, x_ref, g_ref, dinv_ref):
    h = deg_ref[...]
    deg = jnp.sum(h, axis=1, keepdims=True) + 1.0
    dinv = lax.rsqrt(deg)
    dinv_ref[...] = dinv
    g_ref[...] = x_ref[...] * dinv


def _prep_call(deg_parts, xp):
    grid = NPAD // ROW_BLOCK
    return pl.pallas_call(
        _prep_body,
        grid=(grid,),
        in_specs=[
            pl.BlockSpec((ROW_BLOCK, NW), lambda i: (i, 0)),
            pl.BlockSpec((ROW_BLOCK, D_FEAT), lambda i: (i, 0)),
        ],
        out_specs=[
            pl.BlockSpec((ROW_BLOCK, D_FEAT), lambda i: (i, 0)),
            pl.BlockSpec((ROW_BLOCK, 1), lambda i: (i, 0)),
        ],
        out_shape=[
            jax.ShapeDtypeStruct((NPAD, D_FEAT), jnp.float32),
            jax.ShapeDtypeStruct((NPAD, 1), jnp.float32),
        ],
    )(deg_parts, xp)


def _mid_body(acc_ref, g_ref, dinv_ref, w1_ref, b1_ref, w2_ref, g2_ref):
    a = acc_ref[...]
    dinv = dinv_ref[...]
    agg = (a[0] + a[1] + g_ref[...]) * dinv
    h1 = jnp.dot(agg, w1_ref[...], preferred_element_type=jnp.float32,
                 precision=lax.Precision.HIGHEST)
    h1 = jnp.maximum(h1 + b1_ref[...], 0.0)
    p = jnp.dot(h1, w2_ref[...], preferred_element_type=jnp.float32,
                precision=lax.Precision.HIGHEST)
    g2_ref[...] = p * dinv


def _mid_call(acc_parts, g, dinv, W1, b1, W2p):
    grid = NPAD // ROW_BLOCK
    return pl.pallas_call(
        _mid_body,
        grid=(grid,),
        in_specs=[
            pl.BlockSpec((NC, ROW_BLOCK, D_FEAT), lambda i: (0, i, 0)),
            pl.BlockSpec((ROW_BLOCK, D_FEAT), lambda i: (i, 0)),
            pl.BlockSpec((ROW_BLOCK, 1), lambda i: (i, 0)),
            pl.BlockSpec((D_FEAT, HIDDEN), lambda i: (0, 0)),
            pl.BlockSpec((1, HIDDEN), lambda i: (0, 0)),
            pl.BlockSpec((HIDDEN, L2W), lambda i: (0, 0)),
        ],
        out_specs=pl.BlockSpec((ROW_BLOCK, L2W), lambda i: (i, 0)),
        out_shape=jax.ShapeDtypeStruct((NPAD, L2W), jnp.float32),
    )(acc_parts, g, dinv, W1, b1, W2p)


def _fin_body(acc_ref, g2_ref, dinv_ref, b2_ref, out_ref):
    a = acc_ref[...]
    z = (a[0] + a[1] + g2_ref[...]) * dinv_ref[...]
    z = z[:, :N_CLASSES] + b2_ref[...]
    m = jnp.max(z, axis=1, keepdims=True)
    zm = z - m
    lse = jnp.log(jnp.sum(jnp.exp(zm), axis=1, keepdims=True))
    out_ref[...] = zm - lse


def _fin_call(acc2_parts, g2, dinv, b2):
    grid = N_NODES // FIN_BLOCK
    return pl.pallas_call(
        _fin_body,
        grid=(grid,),
        in_specs=[
            pl.BlockSpec((NC, FIN_BLOCK, L2W), lambda i: (0, i, 0)),
            pl.BlockSpec((FIN_BLOCK, L2W), lambda i: (i, 0)),
            pl.BlockSpec((FIN_BLOCK, 1), lambda i: (i, 0)),
            pl.BlockSpec((1, N_CLASSES), lambda i: (0, 0)),
        ],
        out_specs=pl.BlockSpec((FIN_BLOCK, N_CLASSES), lambda i: (i, 0)),
        out_shape=jax.ShapeDtypeStruct((N_NODES, N_CLASSES), jnp.float32),
    )(acc2_parts, g2, dinv, b2)


def kernel(x, edge_index, W1, b1, W2, b2):
    n_extra = E_PAD - N_EDGES
    pad_ids = jnp.arange(n_extra, dtype=jnp.int32)
    src = jnp.concatenate(
        [edge_index[0].astype(jnp.int32), pad_ids % N_NODES]).reshape(NW, NCH, CH)
    dst = jnp.concatenate(
        [edge_index[1].astype(jnp.int32),
         N_NODES + pad_ids % (NPAD - N_NODES)]).reshape(NW, NCH, CH)

    xp = jnp.pad(x, ((0, NPAD - N_NODES), (0, 0)))
    zero_hist = jnp.zeros((NPAD,), jnp.float32)
    zero_128 = jnp.zeros((ROWS_PT, D_FEAT), jnp.float32)
    W2p = jnp.pad(W2, ((0, 0), (0, L2W - N_CLASSES)))
    b1r = b1.reshape(1, HIDDEN)
    b2r = b2.reshape(1, N_CLASSES)

    deg_parts = _deg_kernel(dst, zero_hist)
    deg_t = deg_parts.reshape(NW, NPAD).T

    g, dinv = _prep_call(deg_t, xp)

    acc = _agg_kernel_l1(g, src, dst, zero_128)
    acc = acc.reshape(NC, NPAD, D_FEAT)

    g2 = _mid_call(acc, g, dinv, W1, b1r, W2p)

    acc2 = _agg_kernel_l2(g2, src, dst, zero_128)
    acc2 = acc2.reshape(NC, NPAD, L2W)

    return _fin_call(acc2, g2, dinv, b2r)
